# Initial kernel scaffold; baseline (speedup 1.0000x reference)
import jax
import jax.numpy as jnp
from jax.experimental import pallas as pl


def kernel(u_feat, embed_table, W1, b1, W2, b2):
    raise NotImplementedError("write your pallas kernel here")


# speedup vs baseline: 16.5991x; 16.5991x over previous
import functools

import jax
import jax.numpy as jnp
from jax import lax
from jax.experimental import pallas as pl
from jax.experimental.pallas import tpu as pltpu
from jax.experimental.pallas import tpu_sc as plsc

BATCH = 16384
N_FIELD = 26
EMBED = 16
D_IN = N_FIELD * EMBED
H1 = 256
H2 = 64

NC = 2
NS = 16
NW = NC * NS

N_ROWS = BATCH * N_FIELD
RPW = N_ROWS // NW
N_CHUNKS = 8
CHUNK = RPW // N_CHUNKS


def _gather(idx_flat, table):
    mesh = plsc.VectorSubcoreMesh(core_axis_name="c", subcore_axis_name="s")

    @functools.partial(
        pl.kernel,
        mesh=mesh,
        out_type=jax.ShapeDtypeStruct((N_ROWS, EMBED), jnp.float32),
        compiler_params=pltpu.CompilerParams(use_tc_tiling_on_sc=False),
        scratch_types=[
            pltpu.VMEM((RPW,), jnp.int32),
            pltpu.VMEM((2, CHUNK, EMBED), jnp.float32),
            pltpu.SemaphoreType.DMA,
            pltpu.SemaphoreType.DMA,
            pltpu.SemaphoreType.DMA,
            pltpu.SemaphoreType.DMA,
        ],
    )
    def gather_k(idx_hbm, table_hbm, out_hbm, idx_v, rows_v, sg0, sg1, so0, so1):
        wid = lax.axis_index("s") * NC + lax.axis_index("c")
        sg = [sg0, sg1]
        so = [so0, so1]
        pltpu.sync_copy(idx_hbm.at[pl.ds(wid * RPW, RPW)], idx_v)

        def start_gather(i, buf):
            return pltpu.async_copy(
                table_hbm.at[idx_v.at[pl.ds(i * CHUNK, CHUNK)]],
                rows_v.at[buf], sg[buf])

        def start_out(i, buf):
            return pltpu.async_copy(
                rows_v.at[buf],
                out_hbm.at[pl.ds(wid * RPW + i * CHUNK, CHUNK)],
                so[buf])

        g = start_gather(0, 0)
        outc = [None, None]
        for i in range(N_CHUNKS):
            buf = i & 1
            nxt = None
            if i + 1 < N_CHUNKS:
                if outc[1 - buf] is not None:
                    outc[1 - buf].wait()
                nxt = start_gather(i + 1, 1 - buf)
            g.wait()
            outc[buf] = start_out(i, buf)
            g = nxt
        for oc in outc:
            if oc is not None:
                oc.wait()

    return gather_k(idx_flat, table)


def _mlp_body(emb_ref, w1_ref, b1_ref, w2_ref, b2_ref, out_ref):
    h = jnp.dot(emb_ref[...], w1_ref[...], preferred_element_type=jnp.float32)
    h = jnp.maximum(h + b1_ref[...], 0.0)
    out_ref[...] = (
        jnp.dot(h, w2_ref[...], preferred_element_type=jnp.float32)
        + b2_ref[...])


def _mlp(emb, W1, b1, W2, b2, tb=1024):
    return pl.pallas_call(
        _mlp_body,
        grid=(BATCH // tb,),
        in_specs=[
            pl.BlockSpec((tb, D_IN), lambda i: (i, 0)),
            pl.BlockSpec((D_IN, H1), lambda i: (0, 0)),
            pl.BlockSpec((1, H1), lambda i: (0, 0)),
            pl.BlockSpec((H1, H2), lambda i: (0, 0)),
            pl.BlockSpec((1, H2), lambda i: (0, 0)),
        ],
        out_specs=pl.BlockSpec((tb, H2), lambda i: (i, 0)),
        out_shape=jax.ShapeDtypeStruct((BATCH, H2), jnp.float32),
    )(emb, W1, b1.reshape(1, H1), W2, b2.reshape(1, H2))


def kernel(u_feat, embed_table, W1, b1, W2, b2):
    idx_flat = u_feat.astype(jnp.int32).reshape(N_ROWS)
    gathered = _gather(idx_flat, embed_table)
    emb = gathered.reshape(BATCH, D_IN)
    return _mlp(emb, W1, b1, W2, b2)
